# baseline (speedup 1.0000x reference)
import functools
import math

import jax
import jax.numpy as jnp
from jax import lax
from jax.experimental import pallas as pl
from jax.experimental.pallas import tpu as pltpu
from jax.experimental.pallas import tpu_sc as plsc

D_MODEL = 64
CHUNK = 128
LANES = 16
SCALE = math.sqrt(D_MODEL)


@functools.partial(jax.jit, static_argnames=())
def _run(x2d, table):
    info = plsc.get_sparse_core_info()
    nc, ns = info.num_cores, info.num_subcores
    nw = nc * ns
    total_rows = x2d.shape[0] * x2d.shape[1]
    steps = total_rows // (nw * CHUNK)

    mesh = plsc.VectorSubcoreMesh(core_axis_name="c", subcore_axis_name="s")

    @functools.partial(
        pl.kernel,
        mesh=mesh,
        compiler_params=pltpu.CompilerParams(use_tc_tiling_on_sc=False),
        out_type=jax.ShapeDtypeStruct((total_rows, D_MODEL), jnp.float32),
        scratch_types=[
            pltpu.VMEM((steps, CHUNK), jnp.int32),
            pltpu.VMEM((CHUNK, D_MODEL), jnp.float32),
            pltpu.SemaphoreType.DMA,
        ],
    )
    def emb(x_hbm, table_hbm, out_hbm, idx_v, rows_v, gsem):
        wid = lax.axis_index("s") * nc + lax.axis_index("c")
        row0 = wid * (steps * CHUNK)
        pltpu.sync_copy(x_hbm.at[pl.ds(wid * steps, steps)], idx_v)

        def step(j, carry):
            pltpu.async_copy(table_hbm.at[idx_v.at[j]], rows_v, gsem).wait()

            def scale_row(r, c):
                for l in range(D_MODEL // LANES):
                    s = pl.ds(l * LANES, LANES)
                    rows_v[r, s] = rows_v[r, s] * SCALE
                return c

            lax.fori_loop(0, CHUNK, scale_row, 0)
            pltpu.sync_copy(rows_v, out_hbm.at[pl.ds(row0 + j * CHUNK, CHUNK)])
            return carry

        lax.fori_loop(0, steps, step, 0)

    return emb(x2d, table)


def kernel(x, table):
    b, s = x.shape
    x2d = x.reshape(-1, CHUNK).astype(jnp.int32)
    out = _run(x2d, table)
    return out.reshape(b, s, D_MODEL)


# speedup vs baseline: 1.2066x; 1.2066x over previous
import functools
import math

import jax
import jax.numpy as jnp
from jax import lax
from jax.experimental import pallas as pl
from jax.experimental.pallas import tpu as pltpu
from jax.experimental.pallas import tpu_sc as plsc

D_MODEL = 64
CHUNK = 128
LANES = 16
SCALE = math.sqrt(D_MODEL)
NBUF = 4
PF = 2


@jax.jit
def _run(x2d, table):
    info = plsc.get_sparse_core_info()
    nc, ns = info.num_cores, info.num_subcores
    nw = nc * ns
    total_rows = x2d.shape[0] * x2d.shape[1]
    steps = total_rows // (nw * CHUNK)

    mesh = plsc.VectorSubcoreMesh(core_axis_name="c", subcore_axis_name="s")

    @functools.partial(
        pl.kernel,
        mesh=mesh,
        compiler_params=pltpu.CompilerParams(use_tc_tiling_on_sc=False),
        out_type=jax.ShapeDtypeStruct((total_rows, D_MODEL), jnp.float32),
        scratch_types=(
            [pltpu.VMEM((steps, CHUNK), jnp.int32)]
            + [pltpu.VMEM((CHUNK, D_MODEL), jnp.float32) for _ in range(NBUF)]
            + [pltpu.SemaphoreType.DMA for _ in range(2 * NBUF)]
        ),
    )
    def emb(x_hbm, table_hbm, out_hbm, idx_v, *bufs_and_sems):
        rows = bufs_and_sems[:NBUF]
        gsem = bufs_and_sems[NBUF : 2 * NBUF]
        ssem = bufs_and_sems[2 * NBUF : 3 * NBUF]
        wid = lax.axis_index("s") * nc + lax.axis_index("c")
        row0 = wid * (steps * CHUNK)
        pltpu.sync_copy(x_hbm.at[pl.ds(wid * steps, steps)], idx_v)

        def start_gather(b, j):
            pltpu.make_async_copy(
                table_hbm.at[idx_v.at[j]], rows[b], gsem[b]
            ).start()

        def wait_gather(b):
            pltpu.make_async_copy(
                table_hbm.at[idx_v.at[0]], rows[b], gsem[b]
            ).wait()

        def start_store(b, j):
            pltpu.make_async_copy(
                rows[b], out_hbm.at[pl.ds(row0 + j * CHUNK, CHUNK)], ssem[b]
            ).start()

        def wait_store(b):
            pltpu.make_async_copy(
                rows[b], out_hbm.at[pl.ds(row0, CHUNK)], ssem[b]
            ).wait()

        for b in range(PF):
            start_gather(b, b)

        def outer(g, carry):
            for b in range(NBUF):
                j = g * NBUF + b
                jp = j + PF
                bp = (b + PF) % NBUF

                @pl.when(jp < steps)
                def _():
                    @pl.when(jp >= NBUF)
                    def _():
                        wait_store(bp)

                    start_gather(bp, jp)

                wait_gather(b)

                @plsc.parallel_loop(0, CHUNK, 1, unroll=8)
                def _(r):
                    for l in range(D_MODEL // LANES):
                        s = pl.ds(l * LANES, LANES)
                        rows[b][r, s] = rows[b][r, s] * SCALE

                start_store(b, j)
            return carry

        lax.fori_loop(0, steps // NBUF, outer, 0)

        for b in range(NBUF):
            wait_store(b)

    return emb(x2d, table)


def kernel(x, table):
    b, s = x.shape
    x2d = x.reshape(-1, CHUNK).astype(jnp.int32)
    out = _run(x2d, table)
    return out.reshape(b, s, D_MODEL)
